# baseline (speedup 1.0000x reference)
import jax
import jax.numpy as jnp
from jax import lax
from jax.experimental import pallas as pl
from jax.experimental.pallas import tpu as pltpu
from jax.experimental.pallas import tpu_sc as plsc

_LANES = 16
_N_SC = 13312
_BLOCK_M = 256


def _sc_lookup(ids_flat, table):
    n = ids_flat.shape[0]
    r, d = table.shape
    info = plsc.get_sparse_core_info()
    nw = info.num_cores * info.num_subcores
    rows_per_w = n // nw
    mesh = plsc.VectorSubcoreMesh(core_axis_name="c", subcore_axis_name="s")

    @pl.kernel(
        out_type=jax.ShapeDtypeStruct((n, d), table.dtype),
        mesh=mesh,
        scratch_types=[
            pltpu.VMEM((rows_per_w,), jnp.int32),
            pltpu.VMEM((r, d), jnp.float32),
            pltpu.SemaphoreType.DMA,
        ],
    )
    def k(table_hbm, ids_hbm, out_hbm, idx_v, table_v, sem):
        wid = lax.axis_index("s") * info.num_cores + lax.axis_index("c")
        base = wid * rows_per_w
        pltpu.sync_copy(table_hbm, table_v)
        pltpu.sync_copy(ids_hbm.at[pl.ds(base, rows_per_w)], idx_v)

        @pl.loop(0, rows_per_w, step=_LANES)
        def _(row0):
            v16 = idx_v[pl.ds(row0, _LANES)]
            for i in range(_LANES):
                pltpu.make_async_copy(
                    table_v.at[v16[i]],
                    out_hbm.at[base + row0 + i],
                    sem).start()

        pltpu.make_async_copy(
            out_hbm.at[pl.ds(base, rows_per_w)],
            out_hbm.at[pl.ds(base, rows_per_w)],
            sem).wait()

    return k(table, ids_flat)


def _tc_lookup(ids_flat, table):
    m = ids_flat.shape[0]
    d = table.shape[1]
    ids_f = ids_flat.astype(jnp.float32).reshape(m, 1)

    def body(f_ref, t_ref, o_ref):
        t0 = t_ref[0:1, :]
        t1 = t_ref[1:2, :]
        o_ref[...] = t0 + f_ref[...] * (t1 - t0)

    return pl.pallas_call(
        body,
        grid=(m // _BLOCK_M,),
        in_specs=[
            pl.BlockSpec((_BLOCK_M, 1), lambda i: (i, 0)),
            pl.BlockSpec((3, d), lambda i: (0, 0)),
        ],
        out_specs=pl.BlockSpec((_BLOCK_M, d), lambda i: (i, 0)),
        out_shape=jax.ShapeDtypeStruct((m, d), table.dtype),
    )(ids_f, table)


def kernel(token_type_ids, table):
    b, s = token_type_ids.shape
    d = table.shape[1]
    ids = token_type_ids.reshape(-1)
    out_sc = _sc_lookup(ids[:_N_SC], table)
    out_tc = _tc_lookup(ids[_N_SC:], table)
    out = jnp.concatenate([out_sc, out_tc], axis=0)
    return out.reshape(b, s, d)


# speedup vs baseline: 2.7483x; 2.7483x over previous
import jax
import jax.numpy as jnp
from jax import lax
from jax.experimental import pallas as pl
from jax.experimental.pallas import tpu as pltpu
from jax.experimental.pallas import tpu_sc as plsc

_LANES = 16
_DRAIN = 64


def _sc_lookup(ids_flat, table):
    n = ids_flat.shape[0]
    r, d = table.shape
    info = plsc.get_sparse_core_info()
    nw = info.num_cores * info.num_subcores
    rows_per_w = n // nw
    mesh = plsc.VectorSubcoreMesh(core_axis_name="c", subcore_axis_name="s")

    @pl.kernel(
        out_type=jax.ShapeDtypeStruct((n, d), table.dtype),
        mesh=mesh,
        scratch_types=[
            pltpu.VMEM((rows_per_w,), jnp.int32),
            pltpu.VMEM((r, d), jnp.float32),
            pltpu.SemaphoreType.DMA,
            pltpu.SemaphoreType.DMA,
        ],
    )
    def k(table_hbm, ids_hbm, out_hbm, idx_v, table_v, sem, stage_sem):
        wid = lax.axis_index("s") * info.num_cores + lax.axis_index("c")
        base = wid * rows_per_w
        ct = pltpu.make_async_copy(table_hbm, table_v, stage_sem)
        ci = pltpu.make_async_copy(
            ids_hbm.at[pl.ds(base, rows_per_w)], idx_v, stage_sem)
        ct.start()
        ci.start()
        ct.wait()
        ci.wait()

        @pl.loop(0, rows_per_w, step=2 * _LANES)
        def _(row0):
            for g in range(2):
                v16 = idx_v[pl.ds(row0 + g * _LANES, _LANES)]
                for i in range(_LANES):
                    pltpu.make_async_copy(
                        table_v.at[v16[i]],
                        out_hbm.at[base + row0 + g * _LANES + i],
                        sem).start()

        pltpu.make_async_copy(
            out_hbm.at[pl.ds(base, rows_per_w)],
            out_hbm.at[pl.ds(base, rows_per_w)],
            sem).wait()

    return k(table, ids_flat)


def kernel(token_type_ids, table):
    b, s = token_type_ids.shape
    out = _sc_lookup(token_type_ids.reshape(-1), table)
    return out.reshape(b, s, table.shape[1])


# speedup vs baseline: 2.8074x; 1.0215x over previous
import jax
import jax.numpy as jnp
from jax import lax
from jax.experimental import pallas as pl
from jax.experimental.pallas import tpu as pltpu
from jax.experimental.pallas import tpu_sc as plsc

_LANES = 16
_DRAIN = 64


def _sc_lookup(ids_flat, table):
    n = ids_flat.shape[0]
    r, d = table.shape
    info = plsc.get_sparse_core_info()
    nw = info.num_cores * info.num_subcores
    rows_per_w = n // nw
    mesh = plsc.VectorSubcoreMesh(core_axis_name="c", subcore_axis_name="s")

    @pl.kernel(
        out_type=jax.ShapeDtypeStruct((n, d), table.dtype),
        mesh=mesh,
        scratch_types=[
            pltpu.VMEM((rows_per_w,), jnp.int32),
            pltpu.VMEM((r, d), jnp.float32),
            pltpu.SemaphoreType.DMA,
        ],
    )
    def k(table_hbm, ids_hbm, out_hbm, idx_v, table_v, sem):
        wid = lax.axis_index("s") * info.num_cores + lax.axis_index("c")
        base = wid * rows_per_w
        pltpu.sync_copy(table_hbm, table_v)
        pltpu.sync_copy(ids_hbm.at[pl.ds(base, rows_per_w)], idx_v)

        @pl.loop(0, rows_per_w, step=_LANES)
        def _(row0):
            v16 = idx_v[pl.ds(row0, _LANES)]
            for i in range(_LANES):
                pltpu.make_async_copy(
                    table_v.at[v16[i]],
                    out_hbm.at[base + row0 + i],
                    sem).start()

        pltpu.make_async_copy(
            out_hbm.at[pl.ds(base, rows_per_w)],
            out_hbm.at[pl.ds(base, rows_per_w)],
            sem).wait()

    return k(table, ids_flat)


def kernel(token_type_ids, table):
    b, s = token_type_ids.shape
    out = _sc_lookup(token_type_ids.reshape(-1), table)
    return out.reshape(b, s, table.shape[1])
